# baseline (device time: 42915 ns/iter reference)
import jax
import jax.numpy as jnp
from jax import lax
from jax.experimental import pallas as pl
from jax.experimental.pallas import tpu as pltpu

N_DEV = 16
B = 256
D = 256
BLK = B // N_DEV
G = 4
NG = N_DEV // G


def kernel(x, Win0, Wout0, Win1, Wout1, Win2, Wout2):
    def body(
        x_ref,
        win0_ref,
        wout0_ref,
        win1_ref,
        wout1_ref,
        win2_ref,
        wout2_ref,
        out_ref,
        p_ref,
        r_ref,
        rs_buf,
        ag_buf,
        rs_sems,
        ag_sems,
        rs_send_sems,
        ag_send_sems,
    ):
        my_id = lax.axis_index("i")

        def rs_send_desc(j):
            return pltpu.make_async_remote_copy(
                src_ref=p_ref.at[pl.ds(j * BLK, BLK), :],
                dst_ref=rs_buf.at[my_id],
                send_sem=rs_send_sems.at[j],
                recv_sem=rs_sems.at[my_id],
                device_id=(j,),
                device_id_type=pl.DeviceIdType.MESH,
            )

        def rs_recv_desc(s):
            return pltpu.make_async_remote_copy(
                src_ref=p_ref.at[pl.ds(0, BLK), :],
                dst_ref=rs_buf.at[s],
                send_sem=rs_send_sems.at[s],
                recv_sem=rs_sems.at[s],
                device_id=(s,),
                device_id_type=pl.DeviceIdType.MESH,
            )

        def ag_send_desc(j):
            return pltpu.make_async_remote_copy(
                src_ref=r_ref,
                dst_ref=ag_buf.at[my_id],
                send_sem=ag_send_sems.at[j],
                recv_sem=ag_sems.at[my_id],
                device_id=(j,),
                device_id_type=pl.DeviceIdType.MESH,
            )

        def ag_recv_desc(s):
            return pltpu.make_async_remote_copy(
                src_ref=r_ref,
                dst_ref=ag_buf.at[s],
                send_sem=ag_send_sems.at[s],
                recv_sem=ag_sems.at[s],
                device_id=(s,),
                device_id_type=pl.DeviceIdType.MESH,
            )

        def rs_start_block(s):

            @pl.when(my_id != s)
            def _():
                rs_send_desc(s).start()

            @pl.when(my_id == s)
            def _():
                rs_buf[s, :, :] = p_ref[pl.ds(s * BLK, BLK), :]

        def rs_finish():
            for s in range(N_DEV):

                @pl.when(my_id != s)
                def _(s=s):
                    rs_recv_desc(s).wait_recv()

            R = jnp.sum(rs_buf[:, :, :], axis=0)
            for j in range(N_DEV):

                @pl.when(my_id != j)
                def _(j=j):
                    rs_send_desc(j).wait_send()

            return R

        def ag_start(R, drain_prev):
            if drain_prev:
                for j in range(N_DEV):

                    @pl.when(my_id != j)
                    def _(j=j):
                        ag_send_desc(j).wait_send()

            r_ref[:, :] = R
            for j in range(N_DEV):

                @pl.when(my_id != j)
                def _(j=j):
                    ag_send_desc(j).start()

                @pl.when(my_id == j)
                def _(j=j):
                    ag_buf[j, :, :] = r_ref[:, :]

        def mlp_chunk(xv, win, wout):
            h = jnp.maximum(
                jnp.dot(
                    xv.astype(jnp.bfloat16),
                    win,
                    preferred_element_type=jnp.float32,
                ),
                0.0,
            )
            return jnp.dot(
                h.astype(jnp.bfloat16), wout, preferred_element_type=jnp.float32
            )

        def pipelined_layer(win_ref, wout_ref):
            win = win_ref[:, :].astype(jnp.bfloat16)
            wout = wout_ref[:, :].astype(jnp.bfloat16)
            for g in range(NG):
                for s in range(g * G, (g + 1) * G):

                    @pl.when(my_id != s)
                    def _(s=s):
                        ag_recv_desc(s).wait_recv()

                xg = ag_buf[pl.ds(g * G, G), :, :].reshape(G * BLK, D)
                pg = mlp_chunk(xg, win, wout)
                p_ref[pl.ds(g * G * BLK, G * BLK), :] = pg
                for s in range(g * G, (g + 1) * G):
                    rs_start_block(s)

        p_ref[:, :] = mlp_chunk(
            x_ref[:, :],
            win0_ref[:, :].astype(jnp.bfloat16),
            wout0_ref[:, :].astype(jnp.bfloat16),
        )
        for s in range(N_DEV):
            rs_start_block(s)
        R = rs_finish()

        ag_start(R, drain_prev=False)
        pipelined_layer(win1_ref, wout1_ref)
        R = rs_finish()

        ag_start(R, drain_prev=True)
        pipelined_layer(win2_ref, wout2_ref)
        R = rs_finish()
        out_ref[:, :] = R

        for j in range(N_DEV):

            @pl.when(my_id != j)
            def _(j=j):
                ag_send_desc(j).wait_send()

    return pl.pallas_call(
        body,
        out_shape=jax.ShapeDtypeStruct((BLK, D), jnp.float32),
        in_specs=[pl.BlockSpec(memory_space=pltpu.VMEM)] * 7,
        out_specs=pl.BlockSpec(memory_space=pltpu.VMEM),
        scratch_shapes=[
            pltpu.VMEM((B, D), jnp.float32),
            pltpu.VMEM((BLK, D), jnp.float32),
            pltpu.VMEM((N_DEV, BLK, D), jnp.float32),
            pltpu.VMEM((N_DEV, BLK, D), jnp.float32),
            pltpu.SemaphoreType.DMA((N_DEV,)),
            pltpu.SemaphoreType.DMA((N_DEV,)),
            pltpu.SemaphoreType.DMA((N_DEV,)),
            pltpu.SemaphoreType.DMA((N_DEV,)),
        ],
    )(x, Win0, Wout0, Win1, Wout1, Win2, Wout2)


# device time: 37497 ns/iter; 1.1445x vs baseline; 1.1445x over previous
import jax
import jax.numpy as jnp
from jax import lax
from jax.experimental import pallas as pl
from jax.experimental.pallas import tpu as pltpu

N_DEV = 16
B = 256
D = 256
BLK = B // N_DEV
G = 4
NG = N_DEV // G

CDT = jnp.bfloat16


def _antipode(i: int) -> int:
    z, k = i // 4, i % 4
    return 4 * (3 - z) + ((k + 2) % 4)


def kernel(x, Win0, Wout0, Win1, Wout1, Win2, Wout2):
    def body(
        x_ref,
        win0_ref,
        wout0_ref,
        win1_ref,
        wout1_ref,
        win2_ref,
        wout2_ref,
        out_ref,
        p_ref,
        r_ref,
        rs_buf,
        ag_buf,
        rs_sems,
        ag_sems,
        rs_send_sems,
        ag_send_sems,
    ):
        my_id = lax.axis_index("i")

        def rs_send_desc(b, dest):
            return pltpu.make_async_remote_copy(
                src_ref=p_ref.at[pl.ds(b * BLK, BLK), :],
                dst_ref=rs_buf.at[my_id],
                send_sem=rs_send_sems.at[dest],
                recv_sem=rs_sems.at[my_id],
                device_id=(dest,),
                device_id_type=pl.DeviceIdType.MESH,
            )

        def rs_recv_desc(s):
            return pltpu.make_async_remote_copy(
                src_ref=p_ref.at[pl.ds(0, BLK), :],
                dst_ref=rs_buf.at[s],
                send_sem=rs_send_sems.at[s],
                recv_sem=rs_sems.at[s],
                device_id=(s,),
                device_id_type=pl.DeviceIdType.MESH,
            )

        def ag_send_desc(j):
            return pltpu.make_async_remote_copy(
                src_ref=r_ref,
                dst_ref=ag_buf.at[my_id],
                send_sem=ag_send_sems.at[j],
                recv_sem=ag_sems.at[my_id],
                device_id=(j,),
                device_id_type=pl.DeviceIdType.MESH,
            )

        def ag_recv_desc(s):
            return pltpu.make_async_remote_copy(
                src_ref=r_ref,
                dst_ref=ag_buf.at[s],
                send_sem=ag_send_sems.at[s],
                recv_sem=ag_sems.at[s],
                device_id=(s,),
                device_id_type=pl.DeviceIdType.MESH,
            )

        def rs_start_block(b, dest):

            @pl.when(my_id != dest)
            def _():
                rs_send_desc(b, dest).start()

            @pl.when(my_id == dest)
            def _():
                rs_buf[dest, :, :] = p_ref[pl.ds(b * BLK, BLK), :]

        def rs_finish():
            for s in range(N_DEV):

                @pl.when(my_id != s)
                def _(s=s):
                    rs_recv_desc(s).wait_recv()

            R = jnp.sum(rs_buf[:, :, :].astype(jnp.float32), axis=0)
            for j in range(N_DEV):

                @pl.when(my_id != j)
                def _(j=j):
                    rs_send_desc(0, j).wait_send()

            return R

        def ag_start(R, drain_prev):
            if drain_prev:
                for j in range(N_DEV):

                    @pl.when(my_id != j)
                    def _(j=j):
                        ag_send_desc(j).wait_send()

            r_ref[:, :] = R.astype(CDT)
            for j in range(N_DEV):

                @pl.when(my_id != j)
                def _(j=j):
                    ag_send_desc(j).start()

                @pl.when(my_id == j)
                def _(j=j):
                    ag_buf[j, :, :] = r_ref[:, :]

        def mlp_chunk(xv, win, wout):
            h = jnp.maximum(
                jnp.dot(xv, win, preferred_element_type=jnp.float32), 0.0
            )
            return jnp.dot(
                h.astype(CDT), wout, preferred_element_type=jnp.float32
            )

        def pipelined_layer(win_ref, wout_ref, slot_of_block, dest_of_block):
            win = win_ref[:, :].astype(CDT)
            wout = wout_ref[:, :].astype(CDT)
            for g in range(NG):
                blocks = list(range(g * G, (g + 1) * G))
                for b in blocks:
                    s = slot_of_block(b)

                    @pl.when(my_id != s)
                    def _(s=s):
                        ag_recv_desc(s).wait_recv()

                xg = jnp.concatenate(
                    [ag_buf[slot_of_block(b), :, :] for b in blocks], axis=0
                )
                pg = mlp_chunk(xg, win, wout)
                p_ref[pl.ds(g * G * BLK, G * BLK), :] = pg.astype(CDT)
                for b in blocks:
                    rs_start_block(b, dest_of_block(b))

        p_ref[:, :] = mlp_chunk(
            x_ref[:, :].astype(CDT),
            win0_ref[:, :].astype(CDT),
            wout0_ref[:, :].astype(CDT),
        ).astype(CDT)
        for b in range(N_DEV):
            rs_start_block(b, b)
        R = rs_finish()

        ag_start(R, drain_prev=False)
        pipelined_layer(
            win1_ref, wout1_ref, slot_of_block=lambda b: b,
            dest_of_block=_antipode,
        )
        R = rs_finish()

        ag_start(R, drain_prev=True)
        pipelined_layer(
            win2_ref, wout2_ref, slot_of_block=_antipode,
            dest_of_block=lambda b: b,
        )
        R = rs_finish()
        out_ref[:, :] = R

        for j in range(N_DEV):

            @pl.when(my_id != j)
            def _(j=j):
                ag_send_desc(j).wait_send()

    return pl.pallas_call(
        body,
        out_shape=jax.ShapeDtypeStruct((BLK, D), jnp.float32),
        in_specs=[pl.BlockSpec(memory_space=pltpu.VMEM)] * 7,
        out_specs=pl.BlockSpec(memory_space=pltpu.VMEM),
        scratch_shapes=[
            pltpu.VMEM((B, D), CDT),
            pltpu.VMEM((BLK, D), CDT),
            pltpu.VMEM((N_DEV, BLK, D), CDT),
            pltpu.VMEM((N_DEV, BLK, D), CDT),
            pltpu.SemaphoreType.DMA((N_DEV,)),
            pltpu.SemaphoreType.DMA((N_DEV,)),
            pltpu.SemaphoreType.DMA((N_DEV,)),
            pltpu.SemaphoreType.DMA((N_DEV,)),
        ],
    )(x, Win0, Wout0, Win1, Wout1, Win2, Wout2)
